# baseline (speedup 1.0000x reference)
import functools

import jax
import jax.numpy as jnp
from jax import lax
from jax.experimental import pallas as pl
from jax.experimental.pallas import tpu as pltpu
from jax.experimental.pallas import tpu_sc as plsc

B = 4096
L = 50
LQ = 5
D = 64
DENSE = 128

_NC = 2
_NS = 16
_NW = _NC * _NS
_CH = 128


_NBUF = 2


def _sc_gather(table, idx2d):
  n_chunks = idx2d.shape[0]
  d = table.shape[1]
  n_ch = n_chunks // _NW
  n_grp = n_ch // _NBUF
  assert n_ch * _NW == n_chunks and n_grp * _NBUF == n_ch

  mesh = plsc.VectorSubcoreMesh(core_axis_name="c", subcore_axis_name="s")

  @functools.partial(
      pl.kernel,
      mesh=mesh,
      out_type=jax.ShapeDtypeStruct((n_chunks * _CH, d), jnp.float32),
      scratch_types=[
          pltpu.VMEM((n_ch, _CH), jnp.int32),
          pltpu.VMEM((_NBUF, _CH, d), jnp.float32),
      ] + [pltpu.SemaphoreType.DMA] * _NBUF,
      compiler_params=pltpu.CompilerParams(use_tc_tiling_on_sc=False),
  )
  def k(table_hbm, idx_hbm, out_hbm, idx_v, rows_v, *sems):
    wid = lax.axis_index("s") * _NC + lax.axis_index("c")
    rbase = wid * n_ch
    pltpu.sync_copy(idx_hbm.at[pl.ds(rbase, n_ch)], idx_v)

    def body(g, carry):
      handles = []
      for b in range(_NBUF):
        i = g * _NBUF + b
        handles.append(
            pltpu.async_copy(table_hbm.at[idx_v.at[i]], rows_v.at[b],
                             sems[b]))
      for b in range(_NBUF):
        i = g * _NBUF + b
        handles[b].wait()
        pltpu.sync_copy(rows_v.at[b],
                        out_hbm.at[pl.ds((rbase + i) * _CH, _CH)])
      return carry

    lax.fori_loop(0, n_grp, body, 0, unroll=False)

  return k(table, idx2d)


def _dense_body(
    s_raw, c_raw, b_raw, it_raw, iq_raw,
    src_i, clk_i, brw_i, iq_i, lbl,
    Wti, bti, Wtq, btq,
    Wsq, bsq, qsq, Wsc, bsc, qsc, Wbi, bbi, qbi,
    Wir, bir, Wur, bur, Wua, bua, qua,
    Wiv1, biv1, Wiv2, biv2,
    WuA1, buA1, WuA2, buA2, WiA1, biA1, WiA2, biA2,
    o_bce, o_s1, o_s1i,
):
  pid = pl.program_id(0)
  bb = s_raw.shape[0]
  inv_b = jnp.float32(1.0 / B)

  def attn_pool(raw3, idx, Wt, bt, W, b, q):
    A = jnp.dot(Wt[...], W[...], preferred_element_type=jnp.float32)
    c = jnp.dot(bt[...], W[...], preferred_element_type=jnp.float32) + b[...]
    x2 = raw3.reshape(bb * L, D)
    h = jnp.tanh(jnp.dot(x2, A, preferred_element_type=jnp.float32) + c)
    s = jnp.dot(h, q[...].reshape(DENSE, 1),
                preferred_element_type=jnp.float32).reshape(bb, L)
    s = jnp.where(idx == 0, jnp.float32(-1e9), s)
    a = jax.nn.softmax(s, axis=-1)
    pooled = jnp.sum(a[:, :, None] * raw3, axis=1)
    return jnp.dot(pooled, Wt[...], preferred_element_type=jnp.float32) + bt[...]

  def iv_pool(raw3, idx, ll):
    m = (idx != 0).astype(jnp.float32)
    pooled = jnp.sum(m[:, :, None] * raw3, axis=1)
    cnt = jnp.maximum(jnp.sum(m, axis=1, keepdims=True), 1.0)
    pooled = pooled / cnt
    h = jnp.tanh(jnp.dot(pooled, Wiv1[...],
                         preferred_element_type=jnp.float32) + biv1[...])
    return jnp.tanh(jnp.dot(h, Wiv2[...],
                            preferred_element_type=jnp.float32) + biv2[...])

  def fc_sig(x, W1, b1, W2, b2):
    h = jax.nn.relu(jnp.dot(x, W1[...],
                            preferred_element_type=jnp.float32) + b1[...])
    lg = jnp.sum(h * W2[...], axis=-1, keepdims=True) + b2[...]
    return jax.nn.sigmoid(lg)

  item_emb = jnp.dot(it_raw[...], Wti[...],
                     preferred_element_type=jnp.float32) + bti[...]
  query_rep = attn_pool(s_raw[...], src_i[...], Wtq, btq, Wsq, bsq, qsq)
  click_rep = attn_pool(c_raw[...], clk_i[...], Wti, bti, Wsc, bsc, qsc)
  browse_rep = attn_pool(b_raw[...], brw_i[...], Wti, bti, Wbi, bbi, qbi)

  iv_feats = iv_pool(s_raw[...], src_i[...], L)
  d1 = iv_feats - browse_rep
  s1_part = jnp.sum(d1 * d1) * (inv_b / D)

  uw = fc_sig(jnp.concatenate([iv_feats, browse_rep], axis=-1),
              WuA1, buA1, WuA2, buA2)
  iv_user = uw * iv_feats + (1.0 - uw) * browse_rep

  def u_branch(x):
    u = jnp.tanh(jnp.dot(x, Wur[...], preferred_element_type=jnp.float32)
                 + bur[...])
    hu = jnp.tanh(jnp.dot(u, Wua[...], preferred_element_type=jnp.float32)
                  + bua[...])
    su = jnp.dot(hu, qua[...].reshape(100, 1),
                 preferred_element_type=jnp.float32)
    return u, su

  u0, su0 = u_branch(iv_user)
  u1, su1 = u_branch(query_rep)
  u2, su2 = u_branch(click_rep)
  su = jnp.concatenate([su0, su1, su2], axis=-1)
  au = jax.nn.softmax(su, axis=-1)
  user_rep = (au[:, 0:1] * u0 + au[:, 1:2] * u1 + au[:, 2:3] * u2)

  iv_item = iv_pool(iq_raw[...], iq_i[...], LQ)
  d2 = iv_item - item_emb
  s1i_part = jnp.sum(d2 * d2) * (inv_b / D)

  iw = fc_sig(jnp.concatenate([iv_item, item_emb], axis=-1),
              WiA1, biA1, WiA2, biA2)
  item_rep0 = iw * iv_item + (1.0 - iw) * item_emb
  item_rep = jnp.tanh(jnp.dot(item_rep0, Wir[...],
                              preferred_element_type=jnp.float32) + bir[...])

  logits = jnp.sum(item_rep * user_rep, axis=-1, keepdims=True)
  prob = jnp.clip(jax.nn.sigmoid(logits), 1e-7, 1.0 - 1e-7)
  y = lbl[...]
  bce_part = jnp.sum(-(y * jnp.log(prob) + (1.0 - y) * jnp.log(1.0 - prob))
                     ) * inv_b

  @pl.when(pid == 0)
  def _():
    o_bce[...] = jnp.zeros_like(o_bce)
    o_s1[...] = jnp.zeros_like(o_s1)
    o_s1i[...] = jnp.zeros_like(o_s1i)

  o_bce[...] += bce_part
  o_s1[...] += s1_part
  o_s1i[...] += s1i_part


def _dense(interpret, *args):
  bb = 256
  grid = B // bb

  def full(x):
    return pl.BlockSpec(x.shape, lambda i: (0,) * x.ndim)

  def batched(x):
    return pl.BlockSpec((bb,) + x.shape[1:],
                        lambda i: (i,) + (0,) * (x.ndim - 1))

  (s_raw, c_raw, b_raw, it_raw, iq_raw, src_i, clk_i, brw_i, iq_i, lbl) = (
      args[:10])
  weights = args[10:]
  in_specs = [batched(a) for a in args[:10]] + [full(w) for w in weights]
  out_spec = pl.BlockSpec((1, 1), lambda i: (0, 0))
  return pl.pallas_call(
      _dense_body,
      grid=(grid,),
      in_specs=in_specs,
      out_specs=(out_spec, out_spec, out_spec),
      out_shape=tuple(jax.ShapeDtypeStruct((1, 1), jnp.float32)
                      for _ in range(3)),
      interpret=interpret,
  )(*args)


def kernel(browse_item, src_qry, search_click, item, item_qry, labels,
           item_table, qry_table, Wti, bti, Wtq, btq, Wsq, bsq, qsq,
           Wsc, bsc, qsc, Wbi, bbi, qbi, Wir, bir, Wur, bur, Wua, bua, qua,
           Wiv1, biv1, Wiv2, biv2, WuA1, buA1, WuA2, buA2,
           WiA1, biA1, WiA2, biA2):
  def pad_to(v, n):
    return jnp.concatenate([v, jnp.zeros((n - v.shape[0],), v.dtype)])

  n_item = _NW * 104 * _CH
  n_qry = _NW * 56 * _CH
  item_idx = pad_to(
      jnp.concatenate(
          [browse_item.reshape(-1), search_click.reshape(-1), item]),
      n_item).reshape(-1, _CH)
  qry_idx = pad_to(
      jnp.concatenate([src_qry.reshape(-1), item_qry.reshape(-1)]),
      n_qry).reshape(-1, _CH)

  item_rows = _sc_gather(item_table, item_idx)
  qry_rows = _sc_gather(qry_table, qry_idx)

  b_raw = item_rows[:B * L].reshape(B, L, D)
  c_raw = item_rows[B * L:2 * B * L].reshape(B, L, D)
  it_raw = item_rows[2 * B * L:2 * B * L + B]
  s_raw = qry_rows[:B * L].reshape(B, L, D)
  iq_raw = qry_rows[B * L:B * L + B * LQ].reshape(B, LQ, D)

  r1 = lambda v: v.reshape(1, -1)
  o_bce, o_s1, o_s1i = _dense(
      False,
      s_raw, c_raw, b_raw, it_raw, iq_raw,
      src_qry, search_click, browse_item, item_qry, labels.reshape(B, 1),
      Wti, r1(bti), Wtq, r1(btq),
      Wsq, r1(bsq), r1(qsq), Wsc, r1(bsc), r1(qsc), Wbi, r1(bbi), r1(qbi),
      Wir, r1(bir), Wur, r1(bur), Wua, r1(bua), r1(qua),
      Wiv1, r1(biv1), Wiv2, r1(biv2),
      r1(WuA1) if WuA1.ndim == 1 else WuA1, r1(buA1),
      WuA2.reshape(1, -1), r1(buA2),
      WiA1, r1(biA1), WiA2.reshape(1, -1), r1(biA2),
  )
  return (o_bce[0, 0], o_s1[0, 0], o_s1i[0, 0])


# speedup vs baseline: 1.6162x; 1.6162x over previous
import functools

import jax
import jax.numpy as jnp
from jax import lax
from jax.experimental import pallas as pl
from jax.experimental.pallas import tpu as pltpu
from jax.experimental.pallas import tpu_sc as plsc

B = 4096
L = 50
LQ = 5
D = 64
DENSE = 128

_NC = 2
_NS = 16
_NW = _NC * _NS
_CH = 128


_NBUF = 2


def _sc_gather_multi(table, idxs):
  d = table.shape[1]
  seg_ch = []
  for ix in idxs:
    n_ch = ix.shape[0] // _NW
    assert n_ch * _NW == ix.shape[0]
    seg_ch.append(n_ch)
  tot_ch = sum(seg_ch)

  mesh = plsc.VectorSubcoreMesh(core_axis_name="c", subcore_axis_name="s")

  @functools.partial(
      pl.kernel,
      mesh=mesh,
      out_type=tuple(
          jax.ShapeDtypeStruct((ix.shape[0] * _CH, d), jnp.float32)
          for ix in idxs),
      scratch_types=[
          pltpu.VMEM((tot_ch, _CH), jnp.int32),
          pltpu.VMEM((_NBUF, _CH, d), jnp.float32),
      ] + [pltpu.SemaphoreType.DMA] * _NBUF,
      compiler_params=pltpu.CompilerParams(use_tc_tiling_on_sc=False),
  )
  def k(table_hbm, *refs):
    idx_hbms = refs[:len(idxs)]
    out_hbms = refs[len(idxs):2 * len(idxs)]
    idx_v = refs[2 * len(idxs)]
    rows_v = refs[2 * len(idxs) + 1]
    sems = refs[2 * len(idxs) + 2:]
    wid = lax.axis_index("s") * _NC + lax.axis_index("c")

    soff = 0
    for s, n_ch in enumerate(seg_ch):
      pltpu.sync_copy(idx_hbms[s].at[pl.ds(wid * n_ch, n_ch)],
                      idx_v.at[pl.ds(soff, n_ch)])
      soff += n_ch

    soff = 0
    for s, n_ch in enumerate(seg_ch):
      out = out_hbms[s]
      rbase = wid * n_ch
      n_grp = n_ch // _NBUF

      def body(g, carry, soff=soff, out=out, rbase=rbase):
        handles = []
        for b in range(_NBUF):
          i = g * _NBUF + b
          handles.append(
              pltpu.async_copy(table_hbm.at[idx_v.at[soff + i]],
                               rows_v.at[b], sems[b]))
        for b in range(_NBUF):
          i = g * _NBUF + b
          handles[b].wait()
          pltpu.sync_copy(rows_v.at[b],
                          out.at[pl.ds((rbase + i) * _CH, _CH)])
        return carry

      if n_grp > 0:
        lax.fori_loop(0, n_grp, body, 0, unroll=False)
      for i in range(n_grp * _NBUF, n_ch):
        pltpu.async_copy(table_hbm.at[idx_v.at[soff + i]], rows_v.at[0],
                         sems[0]).wait()
        pltpu.sync_copy(rows_v.at[0], out.at[pl.ds((rbase + i) * _CH, _CH)])
      soff += n_ch

  return k(table, *idxs)


def _dense_body(
    s_raw, c_raw, b_raw, it_raw, iq_raw,
    src_i, clk_i, brw_i, iq_i, lbl,
    Wti, bti, Wtq, btq,
    Wsq, bsq, qsq, Wsc, bsc, qsc, Wbi, bbi, qbi,
    Wir, bir, Wur, bur, Wua, bua, qua,
    Wiv1, biv1, Wiv2, biv2,
    WuA1, buA1, WuA2, buA2, WiA1, biA1, WiA2, biA2,
    o_bce, o_s1, o_s1i,
):
  pid = pl.program_id(0)
  bb = src_i.shape[0]
  inv_b = jnp.float32(1.0 / B)

  def attn_pool(x2, idx, ll, Wt, bt, W, b, q):
    A = jnp.dot(Wt[...], W[...], preferred_element_type=jnp.float32)
    c = jnp.dot(bt[...], W[...], preferred_element_type=jnp.float32) + b[...]
    h = jnp.tanh(jnp.dot(x2, A, preferred_element_type=jnp.float32) + c)
    s = jnp.dot(h, q[...].reshape(DENSE, 1),
                preferred_element_type=jnp.float32).reshape(bb, ll)
    s = jnp.where(idx == 0, jnp.float32(-1e9), s)
    a = jax.nn.softmax(s, axis=-1)
    pooled = jnp.sum(a[:, :, None] * x2.reshape(bb, ll, D), axis=1)
    return jnp.dot(pooled, Wt[...], preferred_element_type=jnp.float32) + bt[...]

  def iv_pool(x2, idx, ll):
    m = (idx != 0).astype(jnp.float32)
    pooled = jnp.sum(m[:, :, None] * x2.reshape(bb, ll, D), axis=1)
    cnt = jnp.maximum(jnp.sum(m, axis=1, keepdims=True), 1.0)
    pooled = pooled / cnt
    h = jnp.tanh(jnp.dot(pooled, Wiv1[...],
                         preferred_element_type=jnp.float32) + biv1[...])
    return jnp.tanh(jnp.dot(h, Wiv2[...],
                            preferred_element_type=jnp.float32) + biv2[...])

  def fc_sig(x, W1, b1, W2, b2):
    h = jax.nn.relu(jnp.dot(x, W1[...],
                            preferred_element_type=jnp.float32) + b1[...])
    lg = jnp.sum(h * W2[...], axis=-1, keepdims=True) + b2[...]
    return jax.nn.sigmoid(lg)

  item_emb = jnp.dot(it_raw[...], Wti[...],
                     preferred_element_type=jnp.float32) + bti[...]
  query_rep = attn_pool(s_raw[...], src_i[...], L, Wtq, btq, Wsq, bsq, qsq)
  click_rep = attn_pool(c_raw[...], clk_i[...], L, Wti, bti, Wsc, bsc, qsc)
  browse_rep = attn_pool(b_raw[...], brw_i[...], L, Wti, bti, Wbi, bbi, qbi)

  iv_feats = iv_pool(s_raw[...], src_i[...], L)
  d1 = iv_feats - browse_rep
  s1_part = jnp.sum(d1 * d1) * (inv_b / D)

  uw = fc_sig(jnp.concatenate([iv_feats, browse_rep], axis=-1),
              WuA1, buA1, WuA2, buA2)
  iv_user = uw * iv_feats + (1.0 - uw) * browse_rep

  def u_branch(x):
    u = jnp.tanh(jnp.dot(x, Wur[...], preferred_element_type=jnp.float32)
                 + bur[...])
    hu = jnp.tanh(jnp.dot(u, Wua[...], preferred_element_type=jnp.float32)
                  + bua[...])
    su = jnp.dot(hu, qua[...].reshape(100, 1),
                 preferred_element_type=jnp.float32)
    return u, su

  u0, su0 = u_branch(iv_user)
  u1, su1 = u_branch(query_rep)
  u2, su2 = u_branch(click_rep)
  su = jnp.concatenate([su0, su1, su2], axis=-1)
  au = jax.nn.softmax(su, axis=-1)
  user_rep = (au[:, 0:1] * u0 + au[:, 1:2] * u1 + au[:, 2:3] * u2)

  iv_item = iv_pool(iq_raw[...], iq_i[...], LQ)
  d2 = iv_item - item_emb
  s1i_part = jnp.sum(d2 * d2) * (inv_b / D)

  iw = fc_sig(jnp.concatenate([iv_item, item_emb], axis=-1),
              WiA1, biA1, WiA2, biA2)
  item_rep0 = iw * iv_item + (1.0 - iw) * item_emb
  item_rep = jnp.tanh(jnp.dot(item_rep0, Wir[...],
                              preferred_element_type=jnp.float32) + bir[...])

  logits = jnp.sum(item_rep * user_rep, axis=-1, keepdims=True)
  prob = jnp.clip(jax.nn.sigmoid(logits), 1e-7, 1.0 - 1e-7)
  y = lbl[...]
  bce_part = jnp.sum(-(y * jnp.log(prob) + (1.0 - y) * jnp.log(1.0 - prob))
                     ) * inv_b

  @pl.when(pid == 0)
  def _():
    o_bce[...] = jnp.zeros_like(o_bce)
    o_s1[...] = jnp.zeros_like(o_s1)
    o_s1i[...] = jnp.zeros_like(o_s1i)

  o_bce[...] += bce_part
  o_s1[...] += s1_part
  o_s1i[...] += s1i_part


def _dense(interpret, *args):
  bb = 128
  grid = B // bb

  def full(x):
    return pl.BlockSpec(x.shape, lambda i: (0,) * x.ndim)

  def rows(x):
    blk = x.shape[0] // grid
    return pl.BlockSpec((blk,) + x.shape[1:],
                        lambda i: (i,) + (0,) * (x.ndim - 1))

  weights = args[10:]
  in_specs = [rows(a) for a in args[:10]] + [full(w) for w in weights]
  out_spec = pl.BlockSpec((1, 1), lambda i: (0, 0))
  return pl.pallas_call(
      _dense_body,
      grid=(grid,),
      in_specs=in_specs,
      out_specs=(out_spec, out_spec, out_spec),
      out_shape=tuple(jax.ShapeDtypeStruct((1, 1), jnp.float32)
                      for _ in range(3)),
      interpret=interpret,
  )(*args)


def kernel(browse_item, src_qry, search_click, item, item_qry, labels,
           item_table, qry_table, Wti, bti, Wtq, btq, Wsq, bsq, qsq,
           Wsc, bsc, qsc, Wbi, bbi, qbi, Wir, bir, Wur, bur, Wua, bua, qua,
           Wiv1, biv1, Wiv2, biv2, WuA1, buA1, WuA2, buA2,
           WiA1, biA1, WiA2, biA2):
  b_raw, c_raw, it_raw = _sc_gather_multi(
      item_table,
      [browse_item.reshape(-1, _CH), search_click.reshape(-1, _CH),
       item.reshape(-1, _CH)])
  s_raw, iq_raw = _sc_gather_multi(
      qry_table,
      [src_qry.reshape(-1, _CH), item_qry.reshape(-1, _CH)])

  r1 = lambda v: v.reshape(1, -1)
  o_bce, o_s1, o_s1i = _dense(
      False,
      s_raw, c_raw, b_raw, it_raw, iq_raw,
      src_qry, search_click, browse_item, item_qry, labels.reshape(B, 1),
      Wti, r1(bti), Wtq, r1(btq),
      Wsq, r1(bsq), r1(qsq), Wsc, r1(bsc), r1(qsc), Wbi, r1(bbi), r1(qbi),
      Wir, r1(bir), Wur, r1(bur), Wua, r1(bua), r1(qua),
      Wiv1, r1(biv1), Wiv2, r1(biv2),
      r1(WuA1) if WuA1.ndim == 1 else WuA1, r1(buA1),
      WuA2.reshape(1, -1), r1(buA2),
      WiA1, r1(biA1), WiA2.reshape(1, -1), r1(biA2),
  )
  return (o_bce[0, 0], o_s1[0, 0], o_s1i[0, 0])
